# Initial kernel scaffold; baseline (speedup 1.0000x reference)
import jax
import jax.numpy as jnp
from jax.experimental import pallas as pl


def kernel(x, edge_index, edge_norm):
    raise NotImplementedError("write your pallas kernel here")


# speedup vs baseline: 6.7689x; 6.7689x over previous
import functools

import jax
import jax.numpy as jnp
from jax import lax
from jax.experimental import pallas as pl
from jax.experimental.pallas import tpu as pltpu
from jax.experimental.pallas import tpu_sc as plsc

N_NODES = 10000
N_PAD = 10240
D = 128
NC = 2
NS = 16
NW = NC * NS
LANES = 16
CHUNK = 128
ROWS_PER_SC_TILE = N_PAD // NS
ZCHUNK = 128
SUPER = 8


def _sc_kernel_body(x_hbm, src_hbm, dst_hbm, norm_hbm, part_hbm,
                    src_v, dst_v, norm_v, rows_v, acc, sem):
    cid = lax.axis_index("c")
    sid = lax.axis_index("s")
    wid = cid * NS + sid
    n_chunks = src_hbm.shape[0] // NW
    n_super = n_chunks // SUPER

    def zrow(e, _):
        for t in range(D // LANES):
            rows_v[e, pl.ds(t * LANES, LANES)] = jnp.zeros((LANES,), jnp.float32)
        return 0
    lax.fori_loop(0, ZCHUNK, zrow, 0)
    row0 = sid * ROWS_PER_SC_TILE
    for q in range(ROWS_PER_SC_TILE // ZCHUNK):
        pltpu.sync_copy(rows_v, acc.at[pl.ds(row0 + q * ZCHUNK, ZCHUNK)])
    plsc.subcore_barrier()

    c0 = wid * n_chunks

    def super_body(s, _):
        pltpu.sync_copy(src_hbm.at[pl.ds(c0 + s * SUPER, SUPER)], src_v)
        pltpu.sync_copy(dst_hbm.at[pl.ds(c0 + s * SUPER, SUPER)], dst_v)
        pltpu.sync_copy(
            norm_hbm.at[pl.ds((c0 + s * SUPER) * CHUNK, SUPER * CHUNK)], norm_v)

        def chunk_body(j, _):
            pltpu.async_copy(x_hbm.at[src_v.at[j]], rows_v, sem).wait()

            def scale(g, _):
                nv16 = norm_v[pl.ds(j * CHUNK + g * LANES, LANES)]
                for l in range(LANES):
                    nv = jnp.full((LANES,), nv16[l])
                    e = g * LANES + l
                    for t in range(D // LANES):
                        sl = pl.ds(t * LANES, LANES)
                        rows_v[e, sl] = rows_v[e, sl] * nv
                return 0
            lax.fori_loop(0, CHUNK // LANES, scale, 0)

            pltpu.sync_copy(rows_v, acc.at[dst_v.at[j]], add=True)
            return 0
        lax.fori_loop(0, SUPER, chunk_body, 0)
        return 0
    lax.fori_loop(0, n_super, super_body, 0)
    plsc.subcore_barrier()

    for q in range(ROWS_PER_SC_TILE // ZCHUNK):
        r = row0 + q * ZCHUNK
        pltpu.sync_copy(acc.at[pl.ds(r, ZCHUNK)], part_hbm.at[cid, pl.ds(r, ZCHUNK)])


def _make_sc_call(n_chunk_rows):
    mesh = plsc.VectorSubcoreMesh(core_axis_name="c", subcore_axis_name="s")
    return pl.kernel(
        _sc_kernel_body,
        mesh=mesh,
        out_type=jax.ShapeDtypeStruct((NC, N_PAD, D), jnp.float32),
        scratch_types=[
            pltpu.VMEM((SUPER, CHUNK), jnp.int32),
            pltpu.VMEM((SUPER, CHUNK), jnp.int32),
            pltpu.VMEM((SUPER * CHUNK,), jnp.float32),
            pltpu.VMEM((CHUNK, D), jnp.float32),
            pltpu.VMEM_SHARED((N_PAD, D), jnp.float32),
            pltpu.SemaphoreType.DMA,
        ],
    )


def _add_body(a_ref, b_ref, o_ref):
    o_ref[...] = a_ref[...] + b_ref[...]


_combine = pl.pallas_call(
    _add_body,
    grid=(10,),
    in_specs=[pl.BlockSpec((N_PAD // 10, D), lambda i: (i, 0))] * 2,
    out_specs=pl.BlockSpec((N_PAD // 10, D), lambda i: (i, 0)),
    out_shape=jax.ShapeDtypeStruct((N_PAD, D), jnp.float32),
)


@jax.jit
def kernel(x, edge_index, edge_norm):
    src = edge_index[0].astype(jnp.int32)
    dst = edge_index[1].astype(jnp.int32)
    norm = edge_norm.reshape(-1).astype(jnp.float32)
    e = src.shape[0]
    per_worker_chunks = -(-e // (NW * CHUNK))
    per_worker_chunks = -(-per_worker_chunks // 8) * 8
    e_pad = per_worker_chunks * NW * CHUNK
    pad = e_pad - e
    if pad:
        fill = (jnp.arange(pad, dtype=jnp.int32) * 37) % N_NODES
        src = jnp.concatenate([src, fill])
        dst = jnp.concatenate([dst, fill])
        norm = jnp.concatenate([norm, jnp.zeros((pad,), jnp.float32)])
    n_chunk_rows = e_pad // CHUNK
    src2 = src.reshape(n_chunk_rows, CHUNK)
    dst2 = dst.reshape(n_chunk_rows, CHUNK)
    part = _make_sc_call(n_chunk_rows)(x, src2, dst2, norm)
    return _combine(part[0], part[1])[:N_NODES]


# speedup vs baseline: 9.2241x; 1.3627x over previous
import functools

import jax
import jax.numpy as jnp
from jax import lax
from jax.experimental import pallas as pl
from jax.experimental.pallas import tpu as pltpu
from jax.experimental.pallas import tpu_sc as plsc

N_NODES = 10000
N_PAD = 10240
D = 128
NC = 2
NS = 16
NW = NC * NS
LANES = 16
CHUNK = 128
ROWS_PER_SC_TILE = N_PAD // NS
ZCHUNK = 128
SUPER = 8


def _sc_kernel_body(x_hbm, src_hbm, dst_hbm, norm_hbm, part_hbm,
                    src_v, dst_v, norm_v, rows_a, rows_b, acc, sem_a, sem_b):
    cid = lax.axis_index("c")
    sid = lax.axis_index("s")
    wid = cid * NS + sid
    n_chunks = src_hbm.shape[0] // NW
    n_super = n_chunks // SUPER

    def zrow(e, _):
        for t in range(D // LANES):
            rows_a[e, pl.ds(t * LANES, LANES)] = jnp.zeros((LANES,), jnp.float32)
        return 0
    lax.fori_loop(0, ZCHUNK, zrow, 0)
    row0 = sid * ROWS_PER_SC_TILE
    for q in range(ROWS_PER_SC_TILE // ZCHUNK):
        pltpu.sync_copy(rows_a, acc.at[pl.ds(row0 + q * ZCHUNK, ZCHUNK)])
    plsc.subcore_barrier()

    c0 = wid * n_chunks

    def gather(j, rows, sem):
        pltpu.async_copy(x_hbm.at[src_v.at[j]], rows, sem)

    def wait_rows(rows, sem):
        pltpu.make_async_copy(x_hbm.at[pl.ds(0, CHUNK)], rows, sem).wait()

    def consume(j, rows):
        def scale(g, _):
            nv16 = norm_v[pl.ds(j * CHUNK + g * LANES, LANES)]
            for l in range(LANES):
                nv = jnp.full((LANES,), nv16[l])
                e = g * LANES + l
                for t in range(D // LANES):
                    sl = pl.ds(t * LANES, LANES)
                    rows[e, sl] = rows[e, sl] * nv
            return 0
        lax.fori_loop(0, CHUNK // LANES, scale, 0)
        pltpu.sync_copy(rows, acc.at[dst_v.at[j]], add=True)

    def super_body(s, _):
        pltpu.sync_copy(src_hbm.at[pl.ds(c0 + s * SUPER, SUPER)], src_v)
        pltpu.sync_copy(dst_hbm.at[pl.ds(c0 + s * SUPER, SUPER)], dst_v)
        pltpu.sync_copy(
            norm_hbm.at[pl.ds((c0 + s * SUPER) * CHUNK, SUPER * CHUNK)], norm_v)

        gather(0, rows_a, sem_a)

        def pair(st, _):
            j0 = st * 2
            wait_rows(rows_a, sem_a)
            gather(j0 + 1, rows_b, sem_b)
            consume(j0, rows_a)
            wait_rows(rows_b, sem_b)

            @pl.when(j0 + 2 < SUPER)
            def _():
                gather(j0 + 2, rows_a, sem_a)
            consume(j0 + 1, rows_b)
            return 0
        lax.fori_loop(0, SUPER // 2, pair, 0)
        return 0
    lax.fori_loop(0, n_super, super_body, 0)
    plsc.subcore_barrier()

    for q in range(ROWS_PER_SC_TILE // ZCHUNK):
        r = row0 + q * ZCHUNK
        pltpu.sync_copy(acc.at[pl.ds(r, ZCHUNK)], part_hbm.at[cid, pl.ds(r, ZCHUNK)])


def _make_sc_call(n_chunk_rows):
    mesh = plsc.VectorSubcoreMesh(core_axis_name="c", subcore_axis_name="s")
    return pl.kernel(
        _sc_kernel_body,
        mesh=mesh,
        out_type=jax.ShapeDtypeStruct((NC, N_PAD, D), jnp.float32),
        scratch_types=[
            pltpu.VMEM((SUPER, CHUNK), jnp.int32),
            pltpu.VMEM((SUPER, CHUNK), jnp.int32),
            pltpu.VMEM((SUPER * CHUNK,), jnp.float32),
            pltpu.VMEM((CHUNK, D), jnp.float32),
            pltpu.VMEM((CHUNK, D), jnp.float32),
            pltpu.VMEM_SHARED((N_PAD, D), jnp.float32),
            pltpu.SemaphoreType.DMA,
            pltpu.SemaphoreType.DMA,
        ],
    )


def _add_body(a_ref, b_ref, o_ref):
    o_ref[...] = a_ref[...] + b_ref[...]


_combine = pl.pallas_call(
    _add_body,
    grid=(10,),
    in_specs=[pl.BlockSpec((N_PAD // 10, D), lambda i: (i, 0))] * 2,
    out_specs=pl.BlockSpec((N_PAD // 10, D), lambda i: (i, 0)),
    out_shape=jax.ShapeDtypeStruct((N_PAD, D), jnp.float32),
)


@jax.jit
def kernel(x, edge_index, edge_norm):
    src = edge_index[0].astype(jnp.int32)
    dst = edge_index[1].astype(jnp.int32)
    norm = edge_norm.reshape(-1).astype(jnp.float32)
    e = src.shape[0]
    per_worker_chunks = -(-e // (NW * CHUNK))
    per_worker_chunks = -(-per_worker_chunks // 8) * 8
    e_pad = per_worker_chunks * NW * CHUNK
    pad = e_pad - e
    if pad:
        fill = (jnp.arange(pad, dtype=jnp.int32) * 37) % N_NODES
        src = jnp.concatenate([src, fill])
        dst = jnp.concatenate([dst, fill])
        norm = jnp.concatenate([norm, jnp.zeros((pad,), jnp.float32)])
    n_chunk_rows = e_pad // CHUNK
    src2 = src.reshape(n_chunk_rows, CHUNK)
    dst2 = dst.reshape(n_chunk_rows, CHUNK)
    part = _make_sc_call(n_chunk_rows)(x, src2, dst2, norm)
    return _combine(part[0], part[1])[:N_NODES]


# speedup vs baseline: 9.2563x; 1.0035x over previous
import functools

import jax
import jax.numpy as jnp
from jax import lax
from jax.experimental import pallas as pl
from jax.experimental.pallas import tpu as pltpu
from jax.experimental.pallas import tpu_sc as plsc

N_NODES = 10000
N_PAD = 10240
D = 128
NC = 2
NS = 16
NW = NC * NS
LANES = 16
CHUNK = 128
ROWS_PER_SC_TILE = N_PAD // NS
ZCHUNK = 128
SUPER = 8
NSPLIT = 2


def _sc_kernel_body(x_hbm, src_hbm, dst_hbm, norm_hbm, part_hbm,
                    src_v, dst_v, norm_v, rows_a, rows_b, acc, sem_a, sem_b):
    cid = lax.axis_index("c")
    sid = lax.axis_index("s")
    wid = cid * NS + sid
    n_chunks = src_hbm.shape[0] // NW
    n_super = n_chunks // SUPER

    def zrow(e, _):
        for t in range(D // LANES):
            rows_a[e, pl.ds(t * LANES, LANES)] = jnp.zeros((LANES,), jnp.float32)
        return 0
    lax.fori_loop(0, ZCHUNK, zrow, 0)
    row0 = sid * ROWS_PER_SC_TILE
    for q in range(ROWS_PER_SC_TILE // ZCHUNK):
        pltpu.sync_copy(rows_a, acc.at[pl.ds(row0 + q * ZCHUNK, ZCHUNK)])
    plsc.subcore_barrier()

    c0 = wid * n_chunks

    def gather(j, rows, sem):
        h = CHUNK // NSPLIT
        for p in range(NSPLIT):
            pltpu.async_copy(x_hbm.at[src_v.at[j, pl.ds(p * h, h)]],
                             rows.at[pl.ds(p * h, h)], sem)

    def wait_rows(rows, sem):
        h = CHUNK // NSPLIT
        for p in range(NSPLIT):
            pltpu.make_async_copy(x_hbm.at[pl.ds(0, h)],
                                  rows.at[pl.ds(p * h, h)], sem).wait()

    def consume(j, rows):
        def scale(g, _):
            nv16 = norm_v[pl.ds(j * CHUNK + g * LANES, LANES)]
            for l in range(LANES):
                nv = jnp.full((LANES,), nv16[l])
                e = g * LANES + l
                for t in range(D // LANES):
                    sl = pl.ds(t * LANES, LANES)
                    rows[e, sl] = rows[e, sl] * nv
            return 0
        lax.fori_loop(0, CHUNK // LANES, scale, 0)
        pltpu.sync_copy(rows, acc.at[dst_v.at[j]], add=True)

    def super_body(s, _):
        pltpu.sync_copy(src_hbm.at[pl.ds(c0 + s * SUPER, SUPER)], src_v)
        pltpu.sync_copy(dst_hbm.at[pl.ds(c0 + s * SUPER, SUPER)], dst_v)
        pltpu.sync_copy(
            norm_hbm.at[pl.ds((c0 + s * SUPER) * CHUNK, SUPER * CHUNK)], norm_v)

        gather(0, rows_a, sem_a)

        def pair(st, _):
            j0 = st * 2
            wait_rows(rows_a, sem_a)
            gather(j0 + 1, rows_b, sem_b)
            consume(j0, rows_a)
            wait_rows(rows_b, sem_b)

            @pl.when(j0 + 2 < SUPER)
            def _():
                gather(j0 + 2, rows_a, sem_a)
            consume(j0 + 1, rows_b)
            return 0
        lax.fori_loop(0, SUPER // 2, pair, 0)
        return 0
    lax.fori_loop(0, n_super, super_body, 0)
    plsc.subcore_barrier()

    for q in range(ROWS_PER_SC_TILE // ZCHUNK):
        r = row0 + q * ZCHUNK
        pltpu.sync_copy(acc.at[pl.ds(r, ZCHUNK)], part_hbm.at[cid, pl.ds(r, ZCHUNK)])


def _make_sc_call(n_chunk_rows):
    mesh = plsc.VectorSubcoreMesh(core_axis_name="c", subcore_axis_name="s")
    return pl.kernel(
        _sc_kernel_body,
        mesh=mesh,
        out_type=jax.ShapeDtypeStruct((NC, N_PAD, D), jnp.float32),
        scratch_types=[
            pltpu.VMEM((SUPER, CHUNK), jnp.int32),
            pltpu.VMEM((SUPER, CHUNK), jnp.int32),
            pltpu.VMEM((SUPER * CHUNK,), jnp.float32),
            pltpu.VMEM((CHUNK, D), jnp.float32),
            pltpu.VMEM((CHUNK, D), jnp.float32),
            pltpu.VMEM_SHARED((N_PAD, D), jnp.float32),
            pltpu.SemaphoreType.DMA,
            pltpu.SemaphoreType.DMA,
        ],
    )


def _add_body(a_ref, b_ref, o_ref):
    o_ref[...] = a_ref[...] + b_ref[...]


_combine = pl.pallas_call(
    _add_body,
    grid=(10,),
    in_specs=[pl.BlockSpec((N_PAD // 10, D), lambda i: (i, 0))] * 2,
    out_specs=pl.BlockSpec((N_PAD // 10, D), lambda i: (i, 0)),
    out_shape=jax.ShapeDtypeStruct((N_PAD, D), jnp.float32),
)


@jax.jit
def kernel(x, edge_index, edge_norm):
    src = edge_index[0].astype(jnp.int32)
    dst = edge_index[1].astype(jnp.int32)
    norm = edge_norm.reshape(-1).astype(jnp.float32)
    e = src.shape[0]
    per_worker_chunks = -(-e // (NW * CHUNK))
    per_worker_chunks = -(-per_worker_chunks // 8) * 8
    e_pad = per_worker_chunks * NW * CHUNK
    pad = e_pad - e
    if pad:
        fill = (jnp.arange(pad, dtype=jnp.int32) * 37) % N_NODES
        src = jnp.concatenate([src, fill])
        dst = jnp.concatenate([dst, fill])
        norm = jnp.concatenate([norm, jnp.zeros((pad,), jnp.float32)])
    n_chunk_rows = e_pad // CHUNK
    src2 = src.reshape(n_chunk_rows, CHUNK)
    dst2 = dst.reshape(n_chunk_rows, CHUNK)
    part = _make_sc_call(n_chunk_rows)(x, src2, dst2, norm)
    return _combine(part[0], part[1])[:N_NODES]
